# Initial kernel scaffold; baseline (speedup 1.0000x reference)
import functools

import jax
import jax.numpy as jnp
from jax import lax
from jax.experimental import pallas as pl
from jax.experimental.pallas import tpu as pltpu
from jax.experimental.pallas import tpu_sc as plsc

_NC, _NS = 2, 16
_NW = _NC * _NS
_CH = 128
_G = 64


def _mesh():
    return plsc.VectorSubcoreMesh(
        core_axis_name="c", subcore_axis_name="s",
        num_cores=_NC, num_subcores=_NS)


def _conv_sc(n_pad, feat, e_tot):
    chunks = e_tot // (_NW * _CH)
    rpt = n_pad // _NS

    @functools.partial(
        pl.kernel,
        out_type=jax.ShapeDtypeStruct((_NC * n_pad, feat), jnp.float32),
        mesh=_mesh(),
        scratch_types=[
            pltpu.VMEM((_CH,), jnp.int32),
            pltpu.VMEM((_CH,), jnp.int32),
            pltpu.VMEM((_CH, feat), jnp.float32),
            pltpu.VMEM_SHARED((n_pad, feat), jnp.float32),
            pltpu.SemaphoreType.DMA,
        ],
    )
    def conv(tab_hbm, src_hbm, dst_hbm, zero_hbm, out_hbm,
             idx_s, idx_d, rows, acc, sem):
        c = lax.axis_index("c")
        s = lax.axis_index("s")
        wid = s * _NC + c
        pltpu.sync_copy(zero_hbm.at[pl.ds(s * rpt, rpt)],
                        acc.at[pl.ds(s * rpt, rpt)])
        plsc.subcore_barrier()
        base = wid * chunks * _CH

        def step(i, carry):
            off = base + i * _CH
            pltpu.sync_copy(src_hbm.at[pl.ds(off, _CH)], idx_s)
            pltpu.sync_copy(dst_hbm.at[pl.ds(off, _CH)], idx_d)
            pltpu.async_copy(tab_hbm.at[idx_s], rows, sem).wait()
            pltpu.sync_copy(rows, acc.at[idx_d], add=True)
            return carry

        lax.fori_loop(0, chunks, step, 0)
        plsc.subcore_barrier()
        pltpu.sync_copy(acc.at[pl.ds(s * rpt, rpt)],
                        out_hbm.at[pl.ds(c * n_pad + s * rpt, rpt)])

    return conv


def _deg_sc(n_pad, e_tot):
    chunks = e_tot // (_NW * _CH)
    rpt = n_pad // _NS

    @functools.partial(
        pl.kernel,
        out_type=jax.ShapeDtypeStruct((_NC * n_pad, 8), jnp.float32),
        mesh=_mesh(),
        scratch_types=[
            pltpu.VMEM((_CH,), jnp.int32),
            pltpu.VMEM((_CH, 8), jnp.float32),
            pltpu.VMEM_SHARED((n_pad, 8), jnp.float32),
        ],
    )
    def deg(dst_hbm, ones_hbm, zero_hbm, out_hbm, idx_d, ones_v, acc):
        c = lax.axis_index("c")
        s = lax.axis_index("s")
        wid = s * _NC + c
        pltpu.sync_copy(ones_hbm, ones_v)
        pltpu.sync_copy(zero_hbm.at[pl.ds(s * rpt, rpt)],
                        acc.at[pl.ds(s * rpt, rpt)])
        plsc.subcore_barrier()
        base = wid * chunks * _CH

        def step(i, carry):
            off = base + i * _CH
            pltpu.sync_copy(dst_hbm.at[pl.ds(off, _CH)], idx_d)
            pltpu.sync_copy(ones_v, acc.at[idx_d], add=True)
            return carry

        lax.fori_loop(0, chunks, step, 0)
        plsc.subcore_barrier()
        pltpu.sync_copy(acc.at[pl.ds(s * rpt, rpt)],
                        out_hbm.at[pl.ds(c * n_pad + s * rpt, rpt)])

    return deg


def _leaky(x):
    return jnp.where(x >= 0, x, 0.2 * x)


def _dinv(d0_ref, d1_ref):
    deg = d0_ref[...][:, :1] + d1_ref[...][:, :1]
    return lax.rsqrt(jnp.maximum(deg, 1.0))


def _tc_first(x_ref, w1_ref, d0_ref, d1_ref, o_ref):
    o_ref[...] = jnp.dot(x_ref[...], w1_ref[...],
                         preferred_element_type=jnp.float32) * _dinv(d0_ref, d1_ref)


def _tc_mid(a0_ref, a1_ref, d0_ref, d1_ref, b1_ref, w2_ref, o_ref):
    dinv = _dinv(d0_ref, d1_ref)
    t = _leaky((a0_ref[...] + a1_ref[...]) * dinv + b1_ref[...])
    o_ref[...] = jnp.dot(t, w2_ref[...],
                         preferred_element_type=jnp.float32) * dinv


def _tc_tail(a0_ref, a1_ref, d0_ref, d1_ref, b2_ref, batch_ref,
             l1w_ref, l1b_ref, l2w_ref, l2b_ref, o_ref):
    dinv = _dinv(d0_ref, d1_ref)
    h = _leaky((a0_ref[...] + a1_ref[...]) * dinv + b2_ref[...])
    n_pad = h.shape[0]
    ids = lax.broadcasted_iota(jnp.int32, (_G, n_pad), 0)
    onehot = (batch_ref[...] == ids).astype(jnp.float32)
    g = jnp.dot(onehot, h, preferred_element_type=jnp.float32)
    g = _leaky(jnp.dot(g, l1w_ref[...],
                       preferred_element_type=jnp.float32) + l1b_ref[...])
    o_ref[...] = jnp.dot(g, l2w_ref[...],
                         preferred_element_type=jnp.float32) + l2b_ref[...]


def kernel(x, edge_index, batch, W1, b1, W2, b2, L1W, L1b, L2W, L2b):
    n, d_in = x.shape
    e = edge_index.shape[1]
    f1 = W1.shape[1]
    f2 = W2.shape[1]
    n_pad = (n + 16) // 16 * 16
    e_real = e + n
    e_tot = -(-e_real // (_NW * _CH)) * (_NW * _CH)

    sl = jnp.arange(n, dtype=jnp.int32)
    pad_idx = jnp.full((e_tot - e_real,), n, dtype=jnp.int32)
    src_all = jnp.concatenate([edge_index[0], sl, pad_idx])
    dst_all = jnp.concatenate([edge_index[1], sl, pad_idx])

    zeros8 = jnp.zeros((n_pad, 8), jnp.float32)
    zeros_f1 = jnp.zeros((n_pad, f1), jnp.float32)
    ones8 = jnp.ones((_CH, 8), jnp.float32)
    x_pad = jnp.pad(x, ((0, n_pad - n), (0, 0)))
    batch_pad = jnp.concatenate(
        [batch, jnp.full((n_pad - n,), _G, jnp.int32)]).reshape(1, n_pad)

    dega = _deg_sc(n_pad, e_tot)(dst_all, ones8, zeros8)
    d0, d1 = dega[:n_pad], dega[n_pad:]

    hp1 = pl.pallas_call(
        _tc_first,
        out_shape=jax.ShapeDtypeStruct((n_pad, f1), jnp.float32),
    )(x_pad, W1, d0, d1)

    acc1 = _conv_sc(n_pad, f1, e_tot)(hp1, src_all, dst_all, zeros_f1)

    hp2 = pl.pallas_call(
        _tc_mid,
        out_shape=jax.ShapeDtypeStruct((n_pad, f2), jnp.float32),
    )(acc1[:n_pad], acc1[n_pad:], d0, d1, b1.reshape(1, f1), W2)

    acc2 = _conv_sc(n_pad, f2, e_tot)(hp2, src_all, dst_all, zeros8[:, :f2])

    out = pl.pallas_call(
        _tc_tail,
        out_shape=jax.ShapeDtypeStruct((_G, L2W.shape[1]), jnp.float32),
    )(acc2[:n_pad], acc2[n_pad:], d0, d1, b2.reshape(1, f2), batch_pad,
      L1W, L1b.reshape(1, -1), L2W, L2b.reshape(1, -1))
    return out


# speedup vs baseline: 20.1041x; 20.1041x over previous
import functools

import jax
import jax.numpy as jnp
from jax import lax
from jax.experimental import pallas as pl
from jax.experimental.pallas import tpu as pltpu
from jax.experimental.pallas import tpu_sc as plsc

_NC, _NS = 2, 16
_NW = _NC * _NS
_CH = 128
_G = 64


def _mesh():
    return plsc.VectorSubcoreMesh(
        core_axis_name="c", subcore_axis_name="s",
        num_cores=_NC, num_subcores=_NS)


def _conv_sc(n_pad, feat, e_tot):
    chunks = e_tot // (_NW * _CH)
    rpt = n_pad // _NS

    @functools.partial(
        pl.kernel,
        out_type=jax.ShapeDtypeStruct((_NC * n_pad, feat), jnp.float32),
        mesh=_mesh(),
        scratch_types=[
            pltpu.VMEM((_CH,), jnp.int32),
            pltpu.VMEM((_CH,), jnp.int32),
            pltpu.VMEM((_CH, feat), jnp.float32),
            pltpu.VMEM_SHARED((n_pad, feat), jnp.float32),
            pltpu.SemaphoreType.DMA,
        ],
        compiler_params=pltpu.CompilerParams(use_tc_tiling_on_sc=False),
    )
    def conv(tab_hbm, src_hbm, dst_hbm, zero_hbm, out_hbm,
             idx_s, idx_d, rows, acc, sem):
        c = lax.axis_index("c")
        s = lax.axis_index("s")
        wid = s * _NC + c
        pltpu.sync_copy(zero_hbm.at[pl.ds(s * rpt, rpt)],
                        acc.at[pl.ds(s * rpt, rpt)])
        plsc.subcore_barrier()
        base = wid * chunks * _CH

        def step(i, carry):
            off = base + i * _CH
            pltpu.sync_copy(src_hbm.at[pl.ds(off, _CH)], idx_s)
            pltpu.sync_copy(dst_hbm.at[pl.ds(off, _CH)], idx_d)
            pltpu.async_copy(tab_hbm.at[idx_s], rows, sem).wait()
            pltpu.sync_copy(rows, acc.at[idx_d], add=True)
            return carry

        lax.fori_loop(0, chunks, step, 0)
        plsc.subcore_barrier()
        pltpu.sync_copy(acc.at[pl.ds(s * rpt, rpt)],
                        out_hbm.at[pl.ds(c * n_pad + s * rpt, rpt)])

    return conv


def _deg_sc(n_pad, e_tot):
    chunks = e_tot // (_NW * _CH)
    rpt = n_pad // _NS

    @functools.partial(
        pl.kernel,
        out_type=jax.ShapeDtypeStruct((_NC * n_pad, 8), jnp.float32),
        mesh=_mesh(),
        scratch_types=[
            pltpu.VMEM((_CH,), jnp.int32),
            pltpu.VMEM((_CH, 8), jnp.float32),
            pltpu.VMEM_SHARED((n_pad, 8), jnp.float32),
        ],
        compiler_params=pltpu.CompilerParams(use_tc_tiling_on_sc=False),
    )
    def deg(dst_hbm, ones_hbm, zero_hbm, out_hbm, idx_d, ones_v, acc):
        c = lax.axis_index("c")
        s = lax.axis_index("s")
        wid = s * _NC + c
        pltpu.sync_copy(ones_hbm, ones_v)
        pltpu.sync_copy(zero_hbm.at[pl.ds(s * rpt, rpt)],
                        acc.at[pl.ds(s * rpt, rpt)])
        plsc.subcore_barrier()
        base = wid * chunks * _CH

        def step(i, carry):
            off = base + i * _CH
            pltpu.sync_copy(dst_hbm.at[pl.ds(off, _CH)], idx_d)
            pltpu.sync_copy(ones_v, acc.at[idx_d], add=True)
            return carry

        lax.fori_loop(0, chunks, step, 0)
        plsc.subcore_barrier()
        pltpu.sync_copy(acc.at[pl.ds(s * rpt, rpt)],
                        out_hbm.at[pl.ds(c * n_pad + s * rpt, rpt)])

    return deg


def _leaky(x):
    return jnp.where(x >= 0, x, 0.2 * x)


def _dinv(d0_ref, d1_ref):
    deg = d0_ref[...][:, :1] + d1_ref[...][:, :1]
    return lax.rsqrt(jnp.maximum(deg, 1.0))


def _tc_first(x_ref, w1_ref, d0_ref, d1_ref, o_ref):
    o_ref[...] = jnp.dot(x_ref[...], w1_ref[...],
                         preferred_element_type=jnp.float32) * _dinv(d0_ref, d1_ref)


def _tc_mid(a0_ref, a1_ref, d0_ref, d1_ref, b1_ref, w2_ref, o_ref):
    dinv = _dinv(d0_ref, d1_ref)
    t = _leaky((a0_ref[...] + a1_ref[...]) * dinv + b1_ref[...])
    o_ref[...] = jnp.dot(t, w2_ref[...],
                         preferred_element_type=jnp.float32) * dinv


def _tc_tail(a0_ref, a1_ref, d0_ref, d1_ref, b2_ref, batch_ref,
             l1w_ref, l1b_ref, l2w_ref, l2b_ref, o_ref):
    dinv = _dinv(d0_ref, d1_ref)
    h = _leaky((a0_ref[...] + a1_ref[...]) * dinv + b2_ref[...])
    n_pad = h.shape[0]
    ids = lax.broadcasted_iota(jnp.int32, (_G, n_pad), 0)
    onehot = (batch_ref[...] == ids).astype(jnp.float32)
    g = jnp.dot(onehot, h, preferred_element_type=jnp.float32)
    g = _leaky(jnp.dot(g, l1w_ref[...],
                       preferred_element_type=jnp.float32) + l1b_ref[...])
    o_ref[...] = jnp.dot(g, l2w_ref[...],
                         preferred_element_type=jnp.float32) + l2b_ref[...]


def kernel(x, edge_index, batch, W1, b1, W2, b2, L1W, L1b, L2W, L2b):
    n, d_in = x.shape
    e = edge_index.shape[1]
    f1 = W1.shape[1]
    f2 = W2.shape[1]
    n_pad = (n + 128) // 128 * 128
    e_real = e + n
    e_tot = -(-e_real // (_NW * _CH)) * (_NW * _CH)

    sl = jnp.arange(n, dtype=jnp.int32)
    pad_idx = jnp.full((e_tot - e_real,), n, dtype=jnp.int32)
    src_all = jnp.concatenate([edge_index[0], sl, pad_idx])
    dst_all = jnp.concatenate([edge_index[1], sl, pad_idx])

    zeros8 = jnp.zeros((n_pad, 8), jnp.float32)
    zeros_f1 = jnp.zeros((n_pad, f1), jnp.float32)
    ones8 = jnp.ones((_CH, 8), jnp.float32)
    x_pad = jnp.pad(x, ((0, n_pad - n), (0, 0)))
    batch_pad = jnp.concatenate(
        [batch, jnp.full((n_pad - n,), _G, jnp.int32)]).reshape(1, n_pad)

    dega = _deg_sc(n_pad, e_tot)(dst_all, ones8, zeros8)
    d0, d1 = dega[:n_pad], dega[n_pad:]

    hp1 = pl.pallas_call(
        _tc_first,
        out_shape=jax.ShapeDtypeStruct((n_pad, f1), jnp.float32),
    )(x_pad, W1, d0, d1)

    acc1 = _conv_sc(n_pad, f1, e_tot)(hp1, src_all, dst_all, zeros_f1)

    hp2 = pl.pallas_call(
        _tc_mid,
        out_shape=jax.ShapeDtypeStruct((n_pad, f2), jnp.float32),
    )(acc1[:n_pad], acc1[n_pad:], d0, d1, b1.reshape(1, f1), W2)

    acc2 = _conv_sc(n_pad, f2, e_tot)(hp2, src_all, dst_all, zeros8[:, :f2])

    out = pl.pallas_call(
        _tc_tail,
        out_shape=jax.ShapeDtypeStruct((_G, L2W.shape[1]), jnp.float32),
    )(acc2[:n_pad], acc2[n_pad:], d0, d1, b2.reshape(1, f2), batch_pad,
      L1W, L1b.reshape(1, -1), L2W, L2b.reshape(1, -1))
    return out


# speedup vs baseline: 25.0892x; 1.2480x over previous
import functools

import jax
import jax.numpy as jnp
from jax import lax
from jax.experimental import pallas as pl
from jax.experimental.pallas import tpu as pltpu
from jax.experimental.pallas import tpu_sc as plsc

_NC, _NS = 2, 16
_NW = _NC * _NS
_CH = 128
_G = 64


def _mesh():
    return plsc.VectorSubcoreMesh(
        core_axis_name="c", subcore_axis_name="s",
        num_cores=_NC, num_subcores=_NS)


def _conv_sc(n_pad, feat, e_tot):
    chunks = e_tot // (_NW * _CH)
    rpt = n_pad // _NS

    @functools.partial(
        pl.kernel,
        out_type=jax.ShapeDtypeStruct((_NC * n_pad, feat), jnp.float32),
        mesh=_mesh(),
        scratch_types=[
            pltpu.VMEM((_CH,), jnp.int32),
            pltpu.VMEM((_CH,), jnp.int32),
            pltpu.VMEM((_CH,), jnp.int32),
            pltpu.VMEM((_CH,), jnp.int32),
            pltpu.VMEM((_CH, feat), jnp.float32),
            pltpu.VMEM((_CH, feat), jnp.float32),
            pltpu.SemaphoreType.DMA,
            pltpu.SemaphoreType.DMA,
            pltpu.SemaphoreType.DMA,
            pltpu.SemaphoreType.DMA,
            pltpu.SemaphoreType.DMA,
            pltpu.VMEM_SHARED((n_pad, feat), jnp.float32),
        ],
        compiler_params=pltpu.CompilerParams(use_tc_tiling_on_sc=False),
    )
    def conv(tab_hbm, src_hbm, dst_hbm, zero_hbm, out_hbm,
             idx_s0, idx_d0, idx_s1, idx_d1, rows0, rows1,
             sem_i0, sem_i1, sem_g, sem_s0, sem_s1, acc):
        c = lax.axis_index("c")
        s = lax.axis_index("s")
        wid = s * _NC + c
        base = wid * chunks * _CH
        last_off = base + (chunks - 1) * _CH
        bufs = ((idx_s0, idx_d0, rows0, sem_i0, sem_s0),
                (idx_s1, idx_d1, rows1, sem_i1, sem_s1))

        def start_idx(g, b):
            off = jnp.minimum(base + g * _CH, last_off)
            pltpu.async_copy(src_hbm.at[pl.ds(off, _CH)], bufs[b][0], bufs[b][3])
            pltpu.async_copy(dst_hbm.at[pl.ds(off, _CH)], bufs[b][1], bufs[b][3])

        def wait_idx(b):
            pltpu.make_async_copy(src_hbm.at[pl.ds(base, _CH)], bufs[b][0],
                                  bufs[b][3]).wait()
            pltpu.make_async_copy(dst_hbm.at[pl.ds(base, _CH)], bufs[b][1],
                                  bufs[b][3]).wait()

        def wait_scatter(b):
            pltpu.make_async_copy(bufs[b][2], acc.at[bufs[b][1]],
                                  bufs[b][4]).wait()

        def chunk(g, b, first):
            idx_s, idx_d, rows, _, sem_s = bufs[b]
            wait_idx(b)
            pltpu.async_copy(tab_hbm.at[idx_s], rows, sem_g).wait()
            if not first:
                wait_scatter(1 - b)
            start_idx(g + 1, 1 - b)
            pltpu.async_copy(rows, acc.at[idx_d], sem_s, add=True)

        pltpu.sync_copy(zero_hbm.at[pl.ds(s * rpt, rpt)],
                        acc.at[pl.ds(s * rpt, rpt)])
        plsc.subcore_barrier()

        start_idx(0, 0)
        chunk(0, 0, True)

        def pair(p, carry):
            g = 1 + 2 * p
            chunk(g, 1, False)
            chunk(g + 1, 0, False)
            return carry

        lax.fori_loop(0, (chunks - 1) // 2, pair, 0)
        wait_scatter(0)
        wait_idx(1)
        plsc.subcore_barrier()
        pltpu.sync_copy(acc.at[pl.ds(s * rpt, rpt)],
                        out_hbm.at[pl.ds(c * n_pad + s * rpt, rpt)])

    return conv


def _deg_sc(n_pad, e_tot):
    chunks = e_tot // (_NW * _CH)
    rpt = n_pad // _NS

    @functools.partial(
        pl.kernel,
        out_type=jax.ShapeDtypeStruct((_NC * n_pad, 8), jnp.float32),
        mesh=_mesh(),
        scratch_types=[
            pltpu.VMEM((_CH,), jnp.int32),
            pltpu.VMEM((_CH,), jnp.int32),
            pltpu.VMEM((_CH, 8), jnp.float32),
            pltpu.SemaphoreType.DMA,
            pltpu.SemaphoreType.DMA,
            pltpu.SemaphoreType.DMA,
            pltpu.SemaphoreType.DMA,
            pltpu.VMEM_SHARED((n_pad, 8), jnp.float32),
        ],
        compiler_params=pltpu.CompilerParams(use_tc_tiling_on_sc=False),
    )
    def deg(dst_hbm, ones_hbm, zero_hbm, out_hbm, idx_d0, idx_d1, ones_v,
            sem_i0, sem_i1, sem_s0, sem_s1, acc):
        c = lax.axis_index("c")
        s = lax.axis_index("s")
        wid = s * _NC + c
        base = wid * chunks * _CH
        last_off = base + (chunks - 1) * _CH
        bufs = ((idx_d0, sem_i0, sem_s0), (idx_d1, sem_i1, sem_s1))

        def start_idx(g, b):
            off = jnp.minimum(base + g * _CH, last_off)
            pltpu.async_copy(dst_hbm.at[pl.ds(off, _CH)], bufs[b][0], bufs[b][1])

        def wait_idx(b):
            pltpu.make_async_copy(dst_hbm.at[pl.ds(base, _CH)], bufs[b][0],
                                  bufs[b][1]).wait()

        def wait_scatter(b):
            pltpu.make_async_copy(ones_v, acc.at[bufs[b][0]], bufs[b][2]).wait()

        def chunk(g, b, first):
            wait_idx(b)
            if not first:
                wait_scatter(1 - b)
            start_idx(g + 1, 1 - b)
            pltpu.async_copy(ones_v, acc.at[bufs[b][0]], bufs[b][2], add=True)

        pltpu.sync_copy(ones_hbm, ones_v)
        pltpu.sync_copy(zero_hbm.at[pl.ds(s * rpt, rpt)],
                        acc.at[pl.ds(s * rpt, rpt)])
        plsc.subcore_barrier()

        start_idx(0, 0)
        chunk(0, 0, True)

        def pair(p, carry):
            g = 1 + 2 * p
            chunk(g, 1, False)
            chunk(g + 1, 0, False)
            return carry

        lax.fori_loop(0, (chunks - 1) // 2, pair, 0)
        wait_scatter(0)
        wait_idx(1)
        plsc.subcore_barrier()
        pltpu.sync_copy(acc.at[pl.ds(s * rpt, rpt)],
                        out_hbm.at[pl.ds(c * n_pad + s * rpt, rpt)])

    return deg


def _leaky(x):
    return jnp.where(x >= 0, x, 0.2 * x)


def _dinv(d0_ref, d1_ref):
    deg = d0_ref[...][:, :1] + d1_ref[...][:, :1]
    return lax.rsqrt(jnp.maximum(deg, 1.0))


def _tc_first(x_ref, w1_ref, d0_ref, d1_ref, o_ref):
    o_ref[...] = jnp.dot(x_ref[...], w1_ref[...],
                         preferred_element_type=jnp.float32) * _dinv(d0_ref, d1_ref)


def _tc_mid(a0_ref, a1_ref, d0_ref, d1_ref, b1_ref, w2_ref, o_ref):
    dinv = _dinv(d0_ref, d1_ref)
    t = _leaky((a0_ref[...] + a1_ref[...]) * dinv + b1_ref[...])
    o_ref[...] = jnp.dot(t, w2_ref[...],
                         preferred_element_type=jnp.float32) * dinv


def _tc_tail(a0_ref, a1_ref, d0_ref, d1_ref, b2_ref, batch_ref,
             l1w_ref, l1b_ref, l2w_ref, l2b_ref, o_ref):
    dinv = _dinv(d0_ref, d1_ref)
    h = _leaky((a0_ref[...] + a1_ref[...]) * dinv + b2_ref[...])
    n_pad = h.shape[0]
    ids = lax.broadcasted_iota(jnp.int32, (_G, n_pad), 0)
    onehot = (batch_ref[...] == ids).astype(jnp.float32)
    g = jnp.dot(onehot, h, preferred_element_type=jnp.float32)
    g = _leaky(jnp.dot(g, l1w_ref[...],
                       preferred_element_type=jnp.float32) + l1b_ref[...])
    o_ref[...] = jnp.dot(g, l2w_ref[...],
                         preferred_element_type=jnp.float32) + l2b_ref[...]


def kernel(x, edge_index, batch, W1, b1, W2, b2, L1W, L1b, L2W, L2b):
    n, d_in = x.shape
    e = edge_index.shape[1]
    f1 = W1.shape[1]
    f2 = W2.shape[1]
    n_pad = (n + 128) // 128 * 128
    e_real = e + n
    e_tot = -(-e_real // (_NW * _CH)) * (_NW * _CH)
    if (e_tot // (_NW * _CH)) % 2 == 0:
        e_tot += _NW * _CH

    sl = jnp.arange(n, dtype=jnp.int32)
    pad_idx = jnp.full((e_tot - e_real,), n, dtype=jnp.int32)
    src_all = jnp.concatenate([edge_index[0], sl, pad_idx])
    dst_all = jnp.concatenate([edge_index[1], sl, pad_idx])

    zeros8 = jnp.zeros((n_pad, 8), jnp.float32)
    zeros_f1 = jnp.zeros((n_pad, f1), jnp.float32)
    ones8 = jnp.ones((_CH, 8), jnp.float32)
    x_pad = jnp.pad(x, ((0, n_pad - n), (0, 0)))
    batch_pad = jnp.concatenate(
        [batch, jnp.full((n_pad - n,), _G, jnp.int32)]).reshape(1, n_pad)

    dega = _deg_sc(n_pad, e_tot)(dst_all, ones8, zeros8)
    d0, d1 = dega[:n_pad], dega[n_pad:]

    hp1 = pl.pallas_call(
        _tc_first,
        out_shape=jax.ShapeDtypeStruct((n_pad, f1), jnp.float32),
    )(x_pad, W1, d0, d1)

    acc1 = _conv_sc(n_pad, f1, e_tot)(hp1, src_all, dst_all, zeros_f1)

    hp2 = pl.pallas_call(
        _tc_mid,
        out_shape=jax.ShapeDtypeStruct((n_pad, f2), jnp.float32),
    )(acc1[:n_pad], acc1[n_pad:], d0, d1, b1.reshape(1, f1), W2)

    acc2 = _conv_sc(n_pad, f2, e_tot)(hp2, src_all, dst_all, zeros8[:, :f2])

    out = pl.pallas_call(
        _tc_tail,
        out_shape=jax.ShapeDtypeStruct((_G, L2W.shape[1]), jnp.float32),
    )(acc2[:n_pad], acc2[n_pad:], d0, d1, b2.reshape(1, f2), batch_pad,
      L1W, L1b.reshape(1, -1), L2W, L2b.reshape(1, -1))
    return out


# speedup vs baseline: 39.7833x; 1.5857x over previous
import functools

import jax
import jax.numpy as jnp
from jax import lax
from jax.experimental import pallas as pl
from jax.experimental.pallas import tpu as pltpu
from jax.experimental.pallas import tpu_sc as plsc

_NC, _NS = 2, 16
_NW = _NC * _NS
_CH = 128
_G = 64


def _mesh():
    return plsc.VectorSubcoreMesh(
        core_axis_name="c", subcore_axis_name="s",
        num_cores=_NC, num_subcores=_NS)


def _conv_sc(n_pad, feat, n_edges):
    ept = n_edges // _NW
    cf = ept // _CH
    tail = ept - cf * _CH
    rpt = n_pad // _NS
    nsteady = cf - 2 - (cf - 2) % 4

    @functools.partial(
        pl.kernel,
        out_type=jax.ShapeDtypeStruct((_NC * n_pad, feat), jnp.float32),
        mesh=_mesh(),
        scratch_types=(
            [pltpu.VMEM((2, _CH), jnp.int32)] * 4
            + [pltpu.VMEM((_CH, feat), jnp.float32)] * 4
            + [pltpu.VMEM((2, max(tail, 8)), jnp.int32),
               pltpu.VMEM((max(tail, 8), feat), jnp.float32)]
            + [pltpu.SemaphoreType.DMA] * 12
            + [pltpu.VMEM_SHARED((n_pad, feat), jnp.float32)]
        ),
        compiler_params=pltpu.CompilerParams(use_tc_tiling_on_sc=False),
    )
    def conv(tab_hbm, ei_hbm, zero_hbm, out_hbm,
             i0, i1, i2, i3, r0, r1, r2, r3, idx_t, rows_t,
             si0, si1, si2, si3, sg0, sg1, sg2, sg3, ss0, ss1, ss2, ss3,
             acc):
        c = lax.axis_index("c")
        s = lax.axis_index("s")
        wid = s * _NC + c
        base = wid * ept
        last_off = base + (cf - 1) * _CH
        idx = (i0, i1, i2, i3)
        rows = (r0, r1, r2, r3)
        sem_i = (si0, si1, si2, si3)
        sem_g = (sg0, sg1, sg2, sg3)
        sem_s = (ss0, ss1, ss2, ss3)

        def start_idx(g, q):
            off = jnp.minimum(base + g * _CH, last_off)
            pltpu.async_copy(ei_hbm.at[:, pl.ds(off, _CH)], idx[q], sem_i[q])

        def wait_idx(q):
            pltpu.make_async_copy(ei_hbm.at[:, pl.ds(base, _CH)], idx[q],
                                  sem_i[q]).wait()

        def start_gather(q):
            pltpu.async_copy(tab_hbm.at[idx[q].at[0]], rows[q], sem_g[q])

        def wait_gather(q):
            pltpu.make_async_copy(tab_hbm.at[idx[q].at[0]], rows[q],
                                  sem_g[q]).wait()

        def start_scatter(q):
            pltpu.async_copy(rows[q], acc.at[idx[q].at[1]], sem_s[q], add=True)

        def wait_scatter(q):
            pltpu.make_async_copy(rows[q], acc.at[idx[q].at[1]],
                                  sem_s[q]).wait()

        def chunk(g, q, skip_scatter_wait):
            wait_gather(q)
            if not skip_scatter_wait:
                wait_scatter((q - 2) % 4)
            start_idx(g + 2, (q + 2) % 4)
            wait_idx((q + 1) % 4)
            start_gather((q + 1) % 4)
            start_scatter(q)

        @pl.when(c == 0)
        def _():
            pltpu.sync_copy(tab_hbm.at[pl.ds(s * rpt, rpt)],
                            acc.at[pl.ds(s * rpt, rpt)])

        @pl.when(c != 0)
        def _():
            pltpu.sync_copy(zero_hbm.at[pl.ds(s * rpt, rpt)],
                            acc.at[pl.ds(s * rpt, rpt)])

        plsc.subcore_barrier()

        start_idx(0, 0)
        start_idx(1, 1)
        wait_idx(0)
        start_gather(0)
        chunk(0, 0, True)
        chunk(1, 1, True)

        def quad(p, carry):
            g = 2 + 4 * p
            for j in range(4):
                chunk(g + j, (2 + j) % 4, False)
            return carry

        lax.fori_loop(0, nsteady // 4, quad, 0)
        for g in range(2 + nsteady, cf):
            chunk(g, g % 4, False)

        wait_scatter((cf - 2) % 4)
        wait_scatter((cf - 1) % 4)
        wait_gather(cf % 4)
        wait_idx((cf + 1) % 4)

        if tail:
            toff = base + cf * _CH
            pltpu.sync_copy(ei_hbm.at[:, pl.ds(toff, tail)],
                            idx_t.at[:, pl.ds(0, tail)])
            pltpu.async_copy(tab_hbm.at[idx_t.at[0, pl.ds(0, tail)]],
                             rows_t.at[pl.ds(0, tail)], sg0).wait()
            pltpu.sync_copy(rows_t.at[pl.ds(0, tail)],
                            acc.at[idx_t.at[1, pl.ds(0, tail)]], add=True)

        plsc.subcore_barrier()
        pltpu.sync_copy(acc.at[pl.ds(s * rpt, rpt)],
                        out_hbm.at[pl.ds(c * n_pad + s * rpt, rpt)])

    return conv


def _deg_sc(n_pad, n_edges):
    ept = n_edges // _NW
    cf = ept // _CH
    tail = ept - cf * _CH
    rpt = n_pad // _NS
    nsteady = cf - 2 - (cf - 2) % 4

    @functools.partial(
        pl.kernel,
        out_type=jax.ShapeDtypeStruct((_NC * n_pad, 8), jnp.float32),
        mesh=_mesh(),
        scratch_types=(
            [pltpu.VMEM((1, _CH), jnp.int32)] * 4
            + [pltpu.VMEM((1, max(tail, 8)), jnp.int32),
               pltpu.VMEM((_CH, 8), jnp.float32)]
            + [pltpu.SemaphoreType.DMA] * 8
            + [pltpu.VMEM_SHARED((n_pad, 8), jnp.float32)]
        ),
        compiler_params=pltpu.CompilerParams(use_tc_tiling_on_sc=False),
    )
    def deg(ei_hbm, ones_hbm, zero_hbm, out_hbm,
            i0, i1, i2, i3, idx_t, ones_v,
            si0, si1, si2, si3, ss0, ss1, ss2, ss3, acc):
        c = lax.axis_index("c")
        s = lax.axis_index("s")
        wid = s * _NC + c
        base = wid * ept
        last_off = base + (cf - 1) * _CH
        idx = (i0, i1, i2, i3)
        sem_i = (si0, si1, si2, si3)
        sem_s = (ss0, ss1, ss2, ss3)

        def start_idx(g, q):
            off = jnp.minimum(base + g * _CH, last_off)
            pltpu.async_copy(ei_hbm.at[pl.ds(1, 1), pl.ds(off, _CH)],
                             idx[q], sem_i[q])

        def wait_idx(q):
            pltpu.make_async_copy(ei_hbm.at[pl.ds(1, 1), pl.ds(base, _CH)],
                                  idx[q], sem_i[q]).wait()

        def start_scatter(q):
            pltpu.async_copy(ones_v, acc.at[idx[q].at[0]], sem_s[q], add=True)

        def wait_scatter(q):
            pltpu.make_async_copy(ones_v, acc.at[idx[q].at[0]],
                                  sem_s[q]).wait()

        def chunk(g, q, skip_scatter_wait):
            wait_idx(q)
            if not skip_scatter_wait:
                wait_scatter((q - 2) % 4)
            start_idx(g + 2, (q + 2) % 4)
            start_scatter(q)

        pltpu.sync_copy(ones_hbm, ones_v)
        pltpu.sync_copy(zero_hbm.at[pl.ds(s * rpt, rpt)],
                        acc.at[pl.ds(s * rpt, rpt)])
        plsc.subcore_barrier()

        start_idx(0, 0)
        start_idx(1, 1)
        chunk(0, 0, True)
        chunk(1, 1, True)

        def quad(p, carry):
            g = 2 + 4 * p
            for j in range(4):
                chunk(g + j, (2 + j) % 4, False)
            return carry

        lax.fori_loop(0, nsteady // 4, quad, 0)
        for g in range(2 + nsteady, cf):
            chunk(g, g % 4, False)

        wait_scatter((cf - 2) % 4)
        wait_scatter((cf - 1) % 4)
        wait_idx(cf % 4)
        wait_idx((cf + 1) % 4)

        if tail:
            toff = base + cf * _CH
            pltpu.sync_copy(ei_hbm.at[pl.ds(1, 1), pl.ds(toff, tail)],
                            idx_t.at[:, pl.ds(0, tail)])
            pltpu.sync_copy(ones_v.at[pl.ds(0, tail)],
                            acc.at[idx_t.at[0, pl.ds(0, tail)]], add=True)

        plsc.subcore_barrier()
        pltpu.sync_copy(acc.at[pl.ds(s * rpt, rpt)],
                        out_hbm.at[pl.ds(c * n_pad + s * rpt, rpt)])

    return deg


def _leaky(x):
    return jnp.where(x >= 0, x, 0.2 * x)


def _dinv(dega_ref, n_pad):
    d = dega_ref[...]
    return lax.rsqrt(d[:n_pad, :1] + d[n_pad:, :1] + 1.0)


def _tc_first(x_ref, w1_ref, dega_ref, o_ref):
    n = x_ref.shape[0]
    n_pad, f1 = o_ref.shape
    dinv = _dinv(dega_ref, n_pad)
    xw = jnp.dot(x_ref[...], w1_ref[...], preferred_element_type=jnp.float32)
    o_ref[pl.ds(0, n), :] = xw * dinv[:n]
    o_ref[pl.ds(n, n_pad - n), :] = jnp.zeros((n_pad - n, f1), jnp.float32)


def _tc_mid(acc_ref, dega_ref, b1_ref, w2_ref, o_ref):
    n_pad = o_ref.shape[0]
    dinv = _dinv(dega_ref, n_pad)
    a = acc_ref[...]
    t = _leaky((a[:n_pad] + a[n_pad:]) * dinv + b1_ref[...])
    o_ref[...] = jnp.dot(t, w2_ref[...],
                         preferred_element_type=jnp.float32) * dinv


def _tc_tail(acc_ref, dega_ref, b2_ref, batch_ref,
             l1w_ref, l1b_ref, l2w_ref, l2b_ref, o_ref):
    n = batch_ref.shape[1]
    n_pad = acc_ref.shape[0] // 2
    dinv = _dinv(dega_ref, n_pad)
    a = acc_ref[...]
    h = _leaky((a[:n_pad] + a[n_pad:]) * dinv + b2_ref[...])[:n]
    ids = lax.broadcasted_iota(jnp.int32, (_G, n), 0)
    onehot = (batch_ref[...] == ids).astype(jnp.float32)
    g = jnp.dot(onehot, h, preferred_element_type=jnp.float32)
    g = _leaky(jnp.dot(g, l1w_ref[...],
                       preferred_element_type=jnp.float32) + l1b_ref[...])
    o_ref[...] = jnp.dot(g, l2w_ref[...],
                         preferred_element_type=jnp.float32) + l2b_ref[...]


def kernel(x, edge_index, batch, W1, b1, W2, b2, L1W, L1b, L2W, L2b):
    n, d_in = x.shape
    e = edge_index.shape[1]
    f1 = W1.shape[1]
    f2 = W2.shape[1]
    n_pad = -(-n // 128) * 128

    zeros_f1 = jnp.zeros((n_pad, f1), jnp.float32)
    ones8 = jnp.ones((_CH, 8), jnp.float32)
    batch2d = batch.reshape(1, n)

    dega = _deg_sc(n_pad, e)(edge_index, ones8, zeros_f1[:, :8])

    hp1 = pl.pallas_call(
        _tc_first,
        out_shape=jax.ShapeDtypeStruct((n_pad, f1), jnp.float32),
    )(x, W1, dega)

    acc1 = _conv_sc(n_pad, f1, e)(hp1, edge_index, zeros_f1)

    hp2 = pl.pallas_call(
        _tc_mid,
        out_shape=jax.ShapeDtypeStruct((n_pad, f2), jnp.float32),
    )(acc1, dega, b1.reshape(1, f1), W2)

    acc2 = _conv_sc(n_pad, f2, e)(hp2, edge_index, zeros_f1[:, :f2])

    out = pl.pallas_call(
        _tc_tail,
        out_shape=jax.ShapeDtypeStruct((_G, L2W.shape[1]), jnp.float32),
    )(acc2, dega, b2.reshape(1, f2), batch2d,
      L1W, L1b.reshape(1, -1), L2W, L2b.reshape(1, -1))
    return out


# speedup vs baseline: 49.9705x; 1.2561x over previous
import functools

import jax
import jax.numpy as jnp
from jax import lax
from jax.experimental import pallas as pl
from jax.experimental.pallas import tpu as pltpu
from jax.experimental.pallas import tpu_sc as plsc

_NC, _NS = 2, 16
_NW = _NC * _NS
_CH = 128
_G = 64


def _mesh():
    return plsc.VectorSubcoreMesh(
        core_axis_name="c", subcore_axis_name="s",
        num_cores=_NC, num_subcores=_NS)


def _conv_sc(n_pad, feat, n_rows):
    cfr = n_rows // _NW
    lr = n_rows - cfr * _NW
    nu = cfr // 2
    rpt = n_pad // _NS
    nsteady = nu - 2 - (nu - 2) % 4

    @functools.partial(
        pl.kernel,
        out_type=jax.ShapeDtypeStruct((_NC * n_pad, feat), jnp.float32),
        mesh=_mesh(),
        scratch_types=(
            [pltpu.VMEM((2, 2, _CH), jnp.int32)] * 4
            + [pltpu.VMEM((_CH, feat), jnp.float32)] * 8
            + [pltpu.SemaphoreType.DMA] * 12
            + [pltpu.VMEM_SHARED((n_pad, feat), jnp.float32)]
        ),
        compiler_params=pltpu.CompilerParams(use_tc_tiling_on_sc=False),
    )
    def conv(tab_hbm, ei_hbm, zero_hbm, out_hbm,
             i0, i1, i2, i3, ra0, ra1, ra2, ra3, rb0, rb1, rb2, rb3,
             si0, si1, si2, si3, sg0, sg1, sg2, sg3, ss0, ss1, ss2, ss3,
             acc):
        c = lax.axis_index("c")
        s = lax.axis_index("s")
        wid = s * _NC + c
        base = wid * cfr
        idx = (i0, i1, i2, i3)
        rows = ((ra0, rb0), (ra1, rb1), (ra2, rb2), (ra3, rb3))
        sem_i = (si0, si1, si2, si3)
        sem_g = (sg0, sg1, sg2, sg3)
        sem_s = (ss0, ss1, ss2, ss3)

        def start_idx(u, q):
            row = base + 2 * jnp.minimum(u, nu - 1)
            pltpu.async_copy(ei_hbm.at[:, pl.ds(row, 2), :], idx[q], sem_i[q])

        def wait_idx(q):
            pltpu.make_async_copy(ei_hbm.at[:, pl.ds(base, 2), :], idx[q],
                                  sem_i[q]).wait()

        def start_gather2(q):
            pltpu.async_copy(tab_hbm.at[idx[q].at[0, 0]], rows[q][0], sem_g[q])
            pltpu.async_copy(tab_hbm.at[idx[q].at[0, 1]], rows[q][1], sem_g[q])

        def wait_gather2(q):
            pltpu.make_async_copy(tab_hbm.at[idx[q].at[0, 0]], rows[q][0],
                                  sem_g[q]).wait()
            pltpu.make_async_copy(tab_hbm.at[idx[q].at[0, 1]], rows[q][1],
                                  sem_g[q]).wait()

        def start_scatter2(q):
            pltpu.async_copy(rows[q][0], acc.at[idx[q].at[1, 0]], sem_s[q],
                             add=True)
            pltpu.async_copy(rows[q][1], acc.at[idx[q].at[1, 1]], sem_s[q],
                             add=True)

        def wait_scatter2(q):
            pltpu.make_async_copy(rows[q][0], acc.at[idx[q].at[1, 0]],
                                  sem_s[q]).wait()
            pltpu.make_async_copy(rows[q][1], acc.at[idx[q].at[1, 1]],
                                  sem_s[q]).wait()

        def unit(u, q, skip_scatter_wait):
            wait_gather2(q)
            if not skip_scatter_wait:
                wait_scatter2((q - 2) % 4)
            start_idx(u + 2, (q + 2) % 4)
            wait_idx((q + 1) % 4)
            start_gather2((q + 1) % 4)
            start_scatter2(q)

        @pl.when(c == 0)
        def _():
            pltpu.sync_copy(tab_hbm.at[pl.ds(s * rpt, rpt)],
                            acc.at[pl.ds(s * rpt, rpt)])

        @pl.when(c != 0)
        def _():
            pltpu.sync_copy(zero_hbm.at[pl.ds(s * rpt, rpt)],
                            acc.at[pl.ds(s * rpt, rpt)])

        plsc.subcore_barrier()

        start_idx(0, 0)
        start_idx(1, 1)
        wait_idx(0)
        start_gather2(0)
        unit(0, 0, True)
        unit(1, 1, True)

        def quad(p, carry):
            u = 2 + 4 * p
            for j in range(4):
                unit(u + j, (2 + j) % 4, False)
            return carry

        lax.fori_loop(0, nsteady // 4, quad, 0)
        for u in range(2 + nsteady, nu):
            unit(u, u % 4, False)

        wait_scatter2((nu - 2) % 4)
        wait_scatter2((nu - 1) % 4)
        wait_gather2(nu % 4)
        wait_idx((nu + 1) % 4)

        if lr:
            @pl.when(wid < lr)
            def _():
                row = _NW * cfr + wid
                pltpu.sync_copy(ei_hbm.at[:, pl.ds(row, 1), :],
                                idx[0].at[:, pl.ds(0, 1), :])
                pltpu.async_copy(tab_hbm.at[idx[0].at[0, 0]], rows[0][0],
                                 sem_g[0]).wait()
                pltpu.sync_copy(rows[0][0], acc.at[idx[0].at[1, 0]], add=True)

        plsc.subcore_barrier()
        pltpu.sync_copy(acc.at[pl.ds(s * rpt, rpt)],
                        out_hbm.at[pl.ds(c * n_pad + s * rpt, rpt)])

    return conv


def _deg_sc(n_pad, n_rows):
    cfr = n_rows // _NW
    lr = n_rows - cfr * _NW
    nu = cfr // 2
    rpt = n_pad // _NS
    nsteady = nu - 2 - (nu - 2) % 4

    @functools.partial(
        pl.kernel,
        out_type=jax.ShapeDtypeStruct((_NC * n_pad, 8), jnp.float32),
        mesh=_mesh(),
        scratch_types=(
            [pltpu.VMEM((1, 2, _CH), jnp.int32)] * 4
            + [pltpu.VMEM((_CH, 8), jnp.float32)]
            + [pltpu.SemaphoreType.DMA] * 8
            + [pltpu.VMEM_SHARED((n_pad, 8), jnp.float32)]
        ),
        compiler_params=pltpu.CompilerParams(use_tc_tiling_on_sc=False),
    )
    def deg(ei_hbm, ones_hbm, zero_hbm, out_hbm,
            i0, i1, i2, i3, ones_v,
            si0, si1, si2, si3, ss0, ss1, ss2, ss3, acc):
        c = lax.axis_index("c")
        s = lax.axis_index("s")
        wid = s * _NC + c
        base = wid * cfr
        idx = (i0, i1, i2, i3)
        sem_i = (si0, si1, si2, si3)
        sem_s = (ss0, ss1, ss2, ss3)

        def start_idx(u, q):
            row = base + 2 * jnp.minimum(u, nu - 1)
            pltpu.async_copy(ei_hbm.at[pl.ds(1, 1), pl.ds(row, 2), :],
                             idx[q], sem_i[q])

        def wait_idx(q):
            pltpu.make_async_copy(ei_hbm.at[pl.ds(1, 1), pl.ds(base, 2), :],
                                  idx[q], sem_i[q]).wait()

        def start_scatter2(q):
            pltpu.async_copy(ones_v, acc.at[idx[q].at[0, 0]], sem_s[q],
                             add=True)
            pltpu.async_copy(ones_v, acc.at[idx[q].at[0, 1]], sem_s[q],
                             add=True)

        def wait_scatter2(q):
            pltpu.make_async_copy(ones_v, acc.at[idx[q].at[0, 0]],
                                  sem_s[q]).wait()
            pltpu.make_async_copy(ones_v, acc.at[idx[q].at[0, 1]],
                                  sem_s[q]).wait()

        def unit(u, q, skip_scatter_wait):
            wait_idx(q)
            if not skip_scatter_wait:
                wait_scatter2((q - 2) % 4)
            start_idx(u + 2, (q + 2) % 4)
            start_scatter2(q)

        pltpu.sync_copy(ones_hbm, ones_v)
        pltpu.sync_copy(zero_hbm.at[pl.ds(s * rpt, rpt)],
                        acc.at[pl.ds(s * rpt, rpt)])
        plsc.subcore_barrier()

        start_idx(0, 0)
        start_idx(1, 1)
        unit(0, 0, True)
        unit(1, 1, True)

        def quad(p, carry):
            u = 2 + 4 * p
            for j in range(4):
                unit(u + j, (2 + j) % 4, False)
            return carry

        lax.fori_loop(0, nsteady // 4, quad, 0)
        for u in range(2 + nsteady, nu):
            unit(u, u % 4, False)

        wait_scatter2((nu - 2) % 4)
        wait_scatter2((nu - 1) % 4)
        wait_idx(nu % 4)
        wait_idx((nu + 1) % 4)

        if lr:
            @pl.when(wid < lr)
            def _():
                row = _NW * cfr + wid
                pltpu.sync_copy(ei_hbm.at[pl.ds(1, 1), pl.ds(row, 1), :],
                                idx[0].at[:, pl.ds(0, 1), :])
                pltpu.sync_copy(ones_v, acc.at[idx[0].at[0, 0]], add=True)

        plsc.subcore_barrier()
        pltpu.sync_copy(acc.at[pl.ds(s * rpt, rpt)],
                        out_hbm.at[pl.ds(c * n_pad + s * rpt, rpt)])

    return deg


def _leaky(x):
    return jnp.where(x >= 0, x, 0.2 * x)


def _dinv(dega_ref, n_pad):
    d = dega_ref[...]
    return lax.rsqrt(d[:n_pad, :1] + d[n_pad:, :1] + 1.0)


def _tc_first(x_ref, w1_ref, dega_ref, o_ref):
    n = x_ref.shape[0]
    n_pad, f1 = o_ref.shape
    dinv = _dinv(dega_ref, n_pad)
    xw = jnp.dot(x_ref[...], w1_ref[...], preferred_element_type=jnp.float32)
    o_ref[pl.ds(0, n), :] = xw * dinv[:n]
    o_ref[pl.ds(n, n_pad - n), :] = jnp.zeros((n_pad - n, f1), jnp.float32)


def _tc_mid(acc_ref, dega_ref, b1_ref, w2_ref, o_ref):
    n_pad = o_ref.shape[0]
    dinv = _dinv(dega_ref, n_pad)
    a = acc_ref[...]
    t = _leaky((a[:n_pad] + a[n_pad:]) * dinv + b1_ref[...])
    o_ref[...] = jnp.dot(t, w2_ref[...],
                         preferred_element_type=jnp.float32) * dinv


def _tc_tail(acc_ref, dega_ref, b2_ref, batch_ref,
             l1w_ref, l1b_ref, l2w_ref, l2b_ref, o_ref):
    n = batch_ref.shape[1]
    n_pad = acc_ref.shape[0] // 2
    dinv = _dinv(dega_ref, n_pad)
    a = acc_ref[...]
    h = _leaky((a[:n_pad] + a[n_pad:]) * dinv + b2_ref[...])[:n]
    ids = lax.broadcasted_iota(jnp.int32, (_G, n), 0)
    onehot = (batch_ref[...] == ids).astype(jnp.float32)
    g = jnp.dot(onehot, h, preferred_element_type=jnp.float32)
    g = _leaky(jnp.dot(g, l1w_ref[...],
                       preferred_element_type=jnp.float32) + l1b_ref[...])
    o_ref[...] = jnp.dot(g, l2w_ref[...],
                         preferred_element_type=jnp.float32) + l2b_ref[...]


def kernel(x, edge_index, batch, W1, b1, W2, b2, L1W, L1b, L2W, L2b):
    n, d_in = x.shape
    e = edge_index.shape[1]
    f1 = W1.shape[1]
    f2 = W2.shape[1]
    n_pad = -(-n // 128) * 128

    zeros_f1 = jnp.zeros((n_pad, f1), jnp.float32)
    ones8 = jnp.ones((_CH, 8), jnp.float32)
    batch2d = batch.reshape(1, n)

    n_rows = e // _CH
    ei3 = edge_index.reshape(2, n_rows, _CH)

    dega = _deg_sc(n_pad, n_rows)(ei3, ones8, zeros_f1[:, :8])

    hp1 = pl.pallas_call(
        _tc_first,
        out_shape=jax.ShapeDtypeStruct((n_pad, f1), jnp.float32),
    )(x, W1, dega)

    acc1 = _conv_sc(n_pad, f1, n_rows)(hp1, ei3, zeros_f1)

    hp2 = pl.pallas_call(
        _tc_mid,
        out_shape=jax.ShapeDtypeStruct((n_pad, f2), jnp.float32),
    )(acc1, dega, b1.reshape(1, f1), W2)

    acc2 = _conv_sc(n_pad, f2, n_rows)(hp2, ei3, zeros_f1[:, :f2])

    out = pl.pallas_call(
        _tc_tail,
        out_shape=jax.ShapeDtypeStruct((_G, L2W.shape[1]), jnp.float32),
    )(acc2, dega, b2.reshape(1, f2), batch2d,
      L1W, L1b.reshape(1, -1), L2W, L2b.reshape(1, -1))
    return out


# speedup vs baseline: 60.2873x; 1.2065x over previous
import functools

import jax
import jax.numpy as jnp
from jax import lax
from jax.experimental import pallas as pl
from jax.experimental.pallas import tpu as pltpu
from jax.experimental.pallas import tpu_sc as plsc

_NC, _NS = 2, 16
_NW = _NC * _NS
_CH = 128
_G = 64


def _mesh():
    return plsc.VectorSubcoreMesh(
        core_axis_name="c", subcore_axis_name="s",
        num_cores=_NC, num_subcores=_NS)


def _conv_sc(n_pad, feat, n_rows):
    cfr = n_rows // _NW
    lr = n_rows - cfr * _NW
    nu = cfr // 2
    rpt = n_pad // _NS
    nsteady = nu - 2 - (nu - 2) % 4

    @functools.partial(
        pl.kernel,
        out_type=jax.ShapeDtypeStruct((_NC * n_pad, feat), jnp.float32),
        mesh=_mesh(),
        scratch_types=(
            [pltpu.VMEM((2, 2, _CH), jnp.int32)] * 4
            + [pltpu.VMEM((_CH, feat), jnp.float32)] * 8
            + [pltpu.SemaphoreType.DMA] * 12
            + [pltpu.VMEM_SHARED((n_pad, feat), jnp.float32)] * 2
        ),
        compiler_params=pltpu.CompilerParams(use_tc_tiling_on_sc=False),
    )
    def conv(tab_hbm, ei_hbm, zero_hbm, out_hbm,
             i0, i1, i2, i3, ra0, ra1, ra2, ra3, rb0, rb1, rb2, rb3,
             si0, si1, si2, si3, sg0, sg1, sg2, sg3, ss0, ss1, ss2, ss3,
             acc, tabs):
        c = lax.axis_index("c")
        s = lax.axis_index("s")
        wid = s * _NC + c
        base = wid * cfr
        idx = (i0, i1, i2, i3)
        rows = ((ra0, rb0), (ra1, rb1), (ra2, rb2), (ra3, rb3))
        sem_i = (si0, si1, si2, si3)
        sem_g = (sg0, sg1, sg2, sg3)
        sem_s = (ss0, ss1, ss2, ss3)

        def start_idx(u, q):
            row = base + 2 * jnp.minimum(u, nu - 1)
            pltpu.async_copy(ei_hbm.at[:, pl.ds(row, 2), :], idx[q], sem_i[q])

        def wait_idx(q):
            pltpu.make_async_copy(ei_hbm.at[:, pl.ds(base, 2), :], idx[q],
                                  sem_i[q]).wait()

        def start_gather2(q):
            pltpu.async_copy(tabs.at[idx[q].at[0, 0]], rows[q][0], sem_g[q])
            pltpu.async_copy(tabs.at[idx[q].at[0, 1]], rows[q][1], sem_g[q])

        def wait_gather2(q):
            pltpu.make_async_copy(tabs.at[idx[q].at[0, 0]], rows[q][0],
                                  sem_g[q]).wait()
            pltpu.make_async_copy(tabs.at[idx[q].at[0, 1]], rows[q][1],
                                  sem_g[q]).wait()

        def start_scatter2(q):
            pltpu.async_copy(rows[q][0], acc.at[idx[q].at[1, 0]], sem_s[q],
                             add=True)
            pltpu.async_copy(rows[q][1], acc.at[idx[q].at[1, 1]], sem_s[q],
                             add=True)

        def wait_scatter2(q):
            pltpu.make_async_copy(rows[q][0], acc.at[idx[q].at[1, 0]],
                                  sem_s[q]).wait()
            pltpu.make_async_copy(rows[q][1], acc.at[idx[q].at[1, 1]],
                                  sem_s[q]).wait()

        def unit(u, q, skip_scatter_wait):
            wait_gather2(q)
            if not skip_scatter_wait:
                wait_scatter2((q - 2) % 4)
            start_idx(u + 2, (q + 2) % 4)
            wait_idx((q + 1) % 4)
            start_gather2((q + 1) % 4)
            start_scatter2(q)

        pltpu.sync_copy(tab_hbm.at[pl.ds(s * rpt, rpt)],
                        tabs.at[pl.ds(s * rpt, rpt)])

        @pl.when(c == 0)
        def _():
            pltpu.sync_copy(tab_hbm.at[pl.ds(s * rpt, rpt)],
                            acc.at[pl.ds(s * rpt, rpt)])

        @pl.when(c != 0)
        def _():
            pltpu.sync_copy(zero_hbm.at[pl.ds(s * rpt, rpt)],
                            acc.at[pl.ds(s * rpt, rpt)])

        plsc.subcore_barrier()

        start_idx(0, 0)
        start_idx(1, 1)
        wait_idx(0)
        start_gather2(0)
        unit(0, 0, True)
        unit(1, 1, True)

        def quad(p, carry):
            u = 2 + 4 * p
            for j in range(4):
                unit(u + j, (2 + j) % 4, False)
            return carry

        lax.fori_loop(0, nsteady // 4, quad, 0)
        for u in range(2 + nsteady, nu):
            unit(u, u % 4, False)

        wait_scatter2((nu - 2) % 4)
        wait_scatter2((nu - 1) % 4)
        wait_gather2(nu % 4)
        wait_idx((nu + 1) % 4)

        if lr:
            @pl.when(wid < lr)
            def _():
                row = _NW * cfr + wid
                pltpu.sync_copy(ei_hbm.at[:, pl.ds(row, 1), :],
                                idx[0].at[:, pl.ds(0, 1), :])
                pltpu.async_copy(tabs.at[idx[0].at[0, 0]], rows[0][0],
                                 sem_g[0]).wait()
                pltpu.sync_copy(rows[0][0], acc.at[idx[0].at[1, 0]], add=True)

        plsc.subcore_barrier()
        pltpu.sync_copy(acc.at[pl.ds(s * rpt, rpt)],
                        out_hbm.at[pl.ds(c * n_pad + s * rpt, rpt)])

    return conv


def _deg_sc(n_pad, n_rows):
    cfr = n_rows // _NW
    lr = n_rows - cfr * _NW
    nu = cfr // 2
    rpt = n_pad // _NS
    nsteady = nu - 2 - (nu - 2) % 4

    @functools.partial(
        pl.kernel,
        out_type=jax.ShapeDtypeStruct((_NC * n_pad, 8), jnp.float32),
        mesh=_mesh(),
        scratch_types=(
            [pltpu.VMEM((1, 2, _CH), jnp.int32)] * 4
            + [pltpu.VMEM((_CH, 8), jnp.float32)]
            + [pltpu.SemaphoreType.DMA] * 8
            + [pltpu.VMEM_SHARED((n_pad, 8), jnp.float32)]
        ),
        compiler_params=pltpu.CompilerParams(use_tc_tiling_on_sc=False),
    )
    def deg(ei_hbm, ones_hbm, zero_hbm, out_hbm,
            i0, i1, i2, i3, ones_v,
            si0, si1, si2, si3, ss0, ss1, ss2, ss3, acc):
        c = lax.axis_index("c")
        s = lax.axis_index("s")
        wid = s * _NC + c
        base = wid * cfr
        idx = (i0, i1, i2, i3)
        sem_i = (si0, si1, si2, si3)
        sem_s = (ss0, ss1, ss2, ss3)

        def start_idx(u, q):
            row = base + 2 * jnp.minimum(u, nu - 1)
            pltpu.async_copy(ei_hbm.at[pl.ds(1, 1), pl.ds(row, 2), :],
                             idx[q], sem_i[q])

        def wait_idx(q):
            pltpu.make_async_copy(ei_hbm.at[pl.ds(1, 1), pl.ds(base, 2), :],
                                  idx[q], sem_i[q]).wait()

        def start_scatter2(q):
            pltpu.async_copy(ones_v, acc.at[idx[q].at[0, 0]], sem_s[q],
                             add=True)
            pltpu.async_copy(ones_v, acc.at[idx[q].at[0, 1]], sem_s[q],
                             add=True)

        def wait_scatter2(q):
            pltpu.make_async_copy(ones_v, acc.at[idx[q].at[0, 0]],
                                  sem_s[q]).wait()
            pltpu.make_async_copy(ones_v, acc.at[idx[q].at[0, 1]],
                                  sem_s[q]).wait()

        def unit(u, q, skip_scatter_wait):
            wait_idx(q)
            if not skip_scatter_wait:
                wait_scatter2((q - 2) % 4)
            start_idx(u + 2, (q + 2) % 4)
            start_scatter2(q)

        pltpu.sync_copy(ones_hbm, ones_v)
        pltpu.sync_copy(zero_hbm.at[pl.ds(s * rpt, rpt)],
                        acc.at[pl.ds(s * rpt, rpt)])
        plsc.subcore_barrier()

        start_idx(0, 0)
        start_idx(1, 1)
        unit(0, 0, True)
        unit(1, 1, True)

        def quad(p, carry):
            u = 2 + 4 * p
            for j in range(4):
                unit(u + j, (2 + j) % 4, False)
            return carry

        lax.fori_loop(0, nsteady // 4, quad, 0)
        for u in range(2 + nsteady, nu):
            unit(u, u % 4, False)

        wait_scatter2((nu - 2) % 4)
        wait_scatter2((nu - 1) % 4)
        wait_idx(nu % 4)
        wait_idx((nu + 1) % 4)

        if lr:
            @pl.when(wid < lr)
            def _():
                row = _NW * cfr + wid
                pltpu.sync_copy(ei_hbm.at[pl.ds(1, 1), pl.ds(row, 1), :],
                                idx[0].at[:, pl.ds(0, 1), :])
                pltpu.sync_copy(ones_v, acc.at[idx[0].at[0, 0]], add=True)

        plsc.subcore_barrier()
        pltpu.sync_copy(acc.at[pl.ds(s * rpt, rpt)],
                        out_hbm.at[pl.ds(c * n_pad + s * rpt, rpt)])

    return deg


def _leaky(x):
    return jnp.where(x >= 0, x, 0.2 * x)


def _dinv(dega_ref, n_pad):
    d = dega_ref[...]
    return lax.rsqrt(d[:n_pad, :1] + d[n_pad:, :1] + 1.0)


def _tc_first(x_ref, w1_ref, dega_ref, o_ref):
    n = x_ref.shape[0]
    n_pad, f1 = o_ref.shape
    dinv = _dinv(dega_ref, n_pad)
    xw = jnp.dot(x_ref[...], w1_ref[...], preferred_element_type=jnp.float32)
    o_ref[pl.ds(0, n), :] = xw * dinv[:n]
    o_ref[pl.ds(n, n_pad - n), :] = jnp.zeros((n_pad - n, f1), jnp.float32)


def _tc_mid(acc_ref, dega_ref, b1_ref, w2_ref, o_ref):
    n_pad = o_ref.shape[0]
    dinv = _dinv(dega_ref, n_pad)
    a = acc_ref[...]
    t = _leaky((a[:n_pad] + a[n_pad:]) * dinv + b1_ref[...])
    o_ref[...] = jnp.dot(t, w2_ref[...],
                         preferred_element_type=jnp.float32) * dinv


def _tc_tail(acc_ref, dega_ref, b2_ref, batch_ref,
             l1w_ref, l1b_ref, l2w_ref, l2b_ref, o_ref):
    n = batch_ref.shape[1]
    n_pad = acc_ref.shape[0] // 2
    dinv = _dinv(dega_ref, n_pad)
    a = acc_ref[...]
    h = _leaky((a[:n_pad] + a[n_pad:]) * dinv + b2_ref[...])[:n]
    ids = lax.broadcasted_iota(jnp.int32, (_G, n), 0)
    onehot = (batch_ref[...] == ids).astype(jnp.float32)
    g = jnp.dot(onehot, h, preferred_element_type=jnp.float32)
    g = _leaky(jnp.dot(g, l1w_ref[...],
                       preferred_element_type=jnp.float32) + l1b_ref[...])
    o_ref[...] = jnp.dot(g, l2w_ref[...],
                         preferred_element_type=jnp.float32) + l2b_ref[...]


def kernel(x, edge_index, batch, W1, b1, W2, b2, L1W, L1b, L2W, L2b):
    n, d_in = x.shape
    e = edge_index.shape[1]
    f1 = W1.shape[1]
    f2 = W2.shape[1]
    n_pad = -(-n // 128) * 128

    zeros_f1 = jnp.zeros((n_pad, f1), jnp.float32)
    ones8 = jnp.ones((_CH, 8), jnp.float32)
    batch2d = batch.reshape(1, n)

    n_rows = e // _CH
    ei3 = edge_index.reshape(2, n_rows, _CH)

    dega = _deg_sc(n_pad, n_rows)(ei3, ones8, zeros_f1[:, :8])

    hp1 = pl.pallas_call(
        _tc_first,
        out_shape=jax.ShapeDtypeStruct((n_pad, f1), jnp.float32),
    )(x, W1, dega)

    acc1 = _conv_sc(n_pad, f1, n_rows)(hp1, ei3, zeros_f1)

    hp2 = pl.pallas_call(
        _tc_mid,
        out_shape=jax.ShapeDtypeStruct((n_pad, f2), jnp.float32),
    )(acc1, dega, b1.reshape(1, f1), W2)

    acc2 = _conv_sc(n_pad, f2, n_rows)(hp2, ei3, zeros_f1[:, :f2])

    out = pl.pallas_call(
        _tc_tail,
        out_shape=jax.ShapeDtypeStruct((_G, L2W.shape[1]), jnp.float32),
    )(acc2, dega, b2.reshape(1, f2), batch2d,
      L1W, L1b.reshape(1, -1), L2W, L2b.reshape(1, -1))
    return out
